# baseline (speedup 1.0000x reference)
import functools

import jax
import jax.numpy as jnp
from jax import lax
from jax.experimental import pallas as pl
from jax.experimental.pallas import tpu as pltpu
from jax.experimental.pallas import tpu_sc as plsc

_B, _N, _H, _M = 1024, 200, 128, 200
_EPS = 1e-12
_T = _B * _N
_NC, _NS, _L = 2, 16, 16
_NW = _NC * _NS
_TW = _T // _NW
_C = 64
_NCHUNK = _TW // _C
_HV = _H // _L


def _sc_body(tbl2_h, p_h, s_h, g_h, b_h, out_h,
             tbl_v, g_v, b_v,
             pidx0, pidx1, sidx0, sidx1,
             rp0, rp1, rs0, rs1, out0, out1,
             semp0, semp1, sems0, sems1, semo0, semo1):
    pidx = [pidx0, pidx1]
    sidx = [sidx0, sidx1]
    rp = [rp0, rp1]
    rs = [rs0, rs1]
    out_v = [out0, out1]
    semp = [semp0, semp1]
    sems = [sems0, sems1]
    semo = [semo0, semo1]

    wid = lax.axis_index("s") * _NC + lax.axis_index("c")
    pltpu.sync_copy(tbl2_h, tbl_v)
    pltpu.sync_copy(g_h, g_v)
    pltpu.sync_copy(b_h, b_v)
    base0 = wid * _TW
    zf = jnp.zeros((_L,), jnp.float32)
    half = jnp.full((_L,), 0.5, jnp.float32)
    magic = jnp.full((_L,), 0x5F3759DF, jnp.int32)
    gs = [g_v[pl.ds(cv * _L, _L)] for cv in range(_HV)]
    bs = [b_v[pl.ds(cv * _L, _L)] for cv in range(_HV)]

    def stage_in(kk, b):
        base = base0 + kk * _C
        pltpu.sync_copy(p_h.at[pl.ds(base, _C)], pidx[b])
        pltpu.sync_copy(s_h.at[pl.ds(base, _C)], sidx[b])
        pltpu.async_copy(tbl2_h.at[pidx[b]], rp[b], semp[b])
        pltpu.async_copy(tbl2_h.at[sidx[b]], rs[b], sems[b])

    for b in range(2):
        stage_in(b, b)

    def chunk_pair(k2, carry):
        for b in range(2):
            kk = k2 * 2 + b
            base = base0 + kk * _C
            pltpu.make_async_copy(tbl2_h.at[pidx[b]], rp[b], semp[b]).wait()
            pltpu.make_async_copy(tbl2_h.at[sidx[b]], rs[b], sems[b]).wait()

            @pl.when(kk >= 2)
            def _wait_out():
                pltpu.make_async_copy(
                    out_v[b], out_h.at[pl.ds(0, _C * _H)], semo[b]).wait()

            rpb, rsb, ovb = rp[b], rs[b], out_v[b]

            @plsc.parallel_loop(0, _C, unroll=2)
            def _tok(t):
                nb = lax.rem(base + t, _N)
                acc = zf
                acc2 = zf
                evs = []
                for cv in range(_HV):
                    vn = tbl_v[nb, pl.ds(cv * _L, _L)]
                    vp = rpb[t, pl.ds(cv * _L, _L)]
                    vs_ = rsb[t, pl.ds(cv * _L, _L)]
                    e = vn + half * (vp + vs_)
                    evs.append(e)
                    acc = acc + e
                    acc2 = acc2 + e * e
                mu_s = jnp.sum(acc) * (1.0 / _H)
                var_s = jnp.sum(acc2) * (1.0 / _H) - mu_s * mu_s + _EPS
                var_v = zf + var_s
                yi = magic - (plsc.bitcast(var_v, jnp.int32) >> 1)
                y = plsc.bitcast(yi, jnp.float32)
                for _ in range(3):
                    y = y * (1.5 - 0.5 * var_v * y * y)
                mu_v = zf + mu_s
                ob = t * _H
                for cv in range(_HV):
                    ovb[pl.ds(ob + cv * _L, _L)] = (
                        (evs[cv] - mu_v) * y * gs[cv] + bs[cv])

            pltpu.async_copy(out_v[b], out_h.at[pl.ds(base * _H, _C * _H)],
                             semo[b])

            @pl.when(kk + 2 < _NCHUNK)
            def _prefetch():
                stage_in(kk + 2, b)
        return carry

    lax.fori_loop(0, _NCHUNK // 2, chunk_pair, 0)
    for b in range(2):
        pltpu.make_async_copy(
            out_v[b], out_h.at[pl.ds(0, _C * _H)], semo[b]).wait()


def kernel(top_vecs, tok_struct_vec, sent_struct_vec, table, gamma, beta):
    del top_vecs, tok_struct_vec
    p_idx = sent_struct_vec[:, :, 0].reshape(_T).astype(jnp.int32)
    s_idx = sent_struct_vec[:, :, 1].reshape(_T).astype(jnp.int32)
    mesh = plsc.VectorSubcoreMesh(core_axis_name="c", subcore_axis_name="s")
    run = functools.partial(
        pl.kernel,
        mesh=mesh,
        compiler_params=pltpu.CompilerParams(needs_layout_passes=False),
        out_type=jax.ShapeDtypeStruct((_T * _H,), jnp.float32),
        scratch_types=[
            pltpu.VMEM((_M, _H), jnp.float32),
            pltpu.VMEM((_H,), jnp.float32),
            pltpu.VMEM((_H,), jnp.float32),
            pltpu.VMEM((_C,), jnp.int32),
            pltpu.VMEM((_C,), jnp.int32),
            pltpu.VMEM((_C,), jnp.int32),
            pltpu.VMEM((_C,), jnp.int32),
            pltpu.VMEM((_C, _H), jnp.float32),
            pltpu.VMEM((_C, _H), jnp.float32),
            pltpu.VMEM((_C, _H), jnp.float32),
            pltpu.VMEM((_C, _H), jnp.float32),
            pltpu.VMEM((_C * _H,), jnp.float32),
            pltpu.VMEM((_C * _H,), jnp.float32),
            pltpu.SemaphoreType.DMA,
            pltpu.SemaphoreType.DMA,
            pltpu.SemaphoreType.DMA,
            pltpu.SemaphoreType.DMA,
            pltpu.SemaphoreType.DMA,
            pltpu.SemaphoreType.DMA,
        ],
    )(_sc_body)
    out = run(table, p_idx, s_idx, gamma, beta)
    return out.reshape(_B, _N, _H)
